# baseline (speedup 1.0000x reference)
import functools

import jax
import jax.numpy as jnp
from jax.experimental import pallas as pl
from jax.experimental.pallas import tpu as pltpu


_BM = 200


def _gcn_kernel(a_ref, x_ref, w_ref, o_ref, xw_ref, *, bm):
    i = pl.program_id(0)

    @pl.when(i == 0)
    def _():
        xw = jnp.dot(x_ref[...], w_ref[...],
                     preferred_element_type=jnp.float32)
        xw_ref[...] = xw.astype(jnp.bfloat16)

    acc = jnp.dot(a_ref[...].astype(jnp.bfloat16), xw_ref[...],
                  preferred_element_type=jnp.float32)
    x_blk = x_ref[pl.ds(i * bm, bm), :]
    mask = jnp.any(x_blk != 0, axis=-1, keepdims=True)
    o_ref[...] = jnp.where(mask, jnp.maximum(acc, 0.0), 0.0)


def kernel(x, a, kernel):
    n, d_in = x.shape[1], x.shape[2]
    d_out = kernel.shape[1]
    x2 = x[0]
    a2 = a[0]

    grid = (n // _BM,)
    out = pl.pallas_call(
        functools.partial(_gcn_kernel, bm=_BM),
        grid=grid,
        in_specs=[
            pl.BlockSpec((_BM, n), lambda i: (i, 0)),
            pl.BlockSpec((n, d_in), lambda i: (0, 0)),
            pl.BlockSpec((d_in, d_out), lambda i: (0, 0)),
        ],
        out_specs=pl.BlockSpec((_BM, d_out), lambda i: (i, 0)),
        out_shape=jax.ShapeDtypeStruct((n, d_out), jnp.float32),
        scratch_shapes=[pltpu.VMEM((n, d_out), jnp.bfloat16)],
    )(a2, x2, kernel)

    return out[None]


# speedup vs baseline: 1.0148x; 1.0148x over previous
import functools

import jax
import jax.numpy as jnp
from jax.experimental import pallas as pl
from jax.experimental.pallas import tpu as pltpu


_BM = 400


def _gcn_kernel(a_ev_ref, a_od_ref, x_ref, w_ref, o_ref, xw_ref, *, bm):
    i = pl.program_id(0)

    @pl.when(i == 0)
    def _():
        xw = jnp.dot(x_ref[...], w_ref[...],
                     preferred_element_type=jnp.float32)
        xw_ref[...] = xw.astype(jnp.bfloat16)

    xw = xw_ref[...]
    acc0 = jnp.dot(a_ev_ref[...].astype(jnp.bfloat16), xw,
                   preferred_element_type=jnp.float32)
    acc1 = jnp.dot(a_od_ref[...].astype(jnp.bfloat16), xw,
                   preferred_element_type=jnp.float32)
    hb = bm // 2
    x_blk = x_ref[pl.ds(i * bm, bm), :]
    mask = jnp.any(x_blk != 0, axis=-1, keepdims=True)
    o_ref[:hb, :] = jnp.where(mask[:hb], jnp.maximum(acc0, 0.0), 0.0)
    o_ref[hb:, :] = jnp.where(mask[hb:], jnp.maximum(acc1, 0.0), 0.0)


def kernel(x, a, kernel):
    n, d_in = x.shape[1], x.shape[2]
    d_out = kernel.shape[1]
    x2 = x[0]
    a2 = a[0]

    hb = _BM // 2
    grid = (n // _BM,)
    out = pl.pallas_call(
        functools.partial(_gcn_kernel, bm=_BM),
        grid=grid,
        in_specs=[
            pl.BlockSpec((hb, n), lambda i: (2 * i, 0)),
            pl.BlockSpec((hb, n), lambda i: (2 * i + 1, 0)),
            pl.BlockSpec((n, d_in), lambda i: (0, 0)),
            pl.BlockSpec((d_in, d_out), lambda i: (0, 0)),
        ],
        out_specs=pl.BlockSpec((_BM, d_out), lambda i: (i, 0)),
        out_shape=jax.ShapeDtypeStruct((n, d_out), jnp.float32),
        scratch_shapes=[pltpu.VMEM((n, d_out), jnp.bfloat16)],
    )(a2, a2, x2, kernel)

    return out[None]


# speedup vs baseline: 1.0154x; 1.0006x over previous
import functools

import jax
import jax.numpy as jnp
from jax.experimental import pallas as pl
from jax.experimental.pallas import tpu as pltpu


_BM = 400


def _gcn_kernel(a_ref, x_ref, w_ref, o_ref, xw_ref, *, bm):
    i = pl.program_id(0)

    @pl.when(i == 0)
    def _():
        xw = jnp.dot(x_ref[...].astype(jnp.bfloat16),
                     w_ref[...].astype(jnp.bfloat16),
                     preferred_element_type=jnp.float32)
        xw_ref[...] = xw.astype(jnp.bfloat16)

    acc = jnp.dot(a_ref[...].astype(jnp.bfloat16), xw_ref[...],
                  preferred_element_type=jnp.float32)
    x_blk = x_ref[pl.ds(i * bm, bm), :]
    mask = jnp.any(x_blk != 0, axis=-1, keepdims=True)
    o_ref[...] = jnp.where(mask, jnp.maximum(acc, 0.0), 0.0)


def kernel(x, a, kernel):
    n, d_in = x.shape[1], x.shape[2]
    d_out = kernel.shape[1]
    x2 = x[0]
    a2 = a[0]

    grid = (n // _BM,)
    out = pl.pallas_call(
        functools.partial(_gcn_kernel, bm=_BM),
        grid=grid,
        in_specs=[
            pl.BlockSpec((_BM, n), lambda i: (i, 0)),
            pl.BlockSpec((n, d_in), lambda i: (0, 0)),
            pl.BlockSpec((d_in, d_out), lambda i: (0, 0)),
        ],
        out_specs=pl.BlockSpec((_BM, d_out), lambda i: (i, 0)),
        out_shape=jax.ShapeDtypeStruct((n, d_out), jnp.float32),
        scratch_shapes=[pltpu.VMEM((n, d_out), jnp.bfloat16)],
    )(a2, x2, kernel)

    return out[None]
